# baseline (speedup 1.0000x reference)
import functools
import math

import jax
import jax.numpy as jnp
from jax import lax
from jax.experimental import pallas as pl
from jax.experimental.pallas import tpu as pltpu
from jax.experimental.pallas import tpu_sc as plsc

_K = 8
_BSZ = 128
_N = 1000
_ROWS = 16
_SC_LANES = 16
_SC_WORKERS = _BSZ // _SC_LANES


def _expm1(x):
    u = jnp.exp(x)
    um1 = u - 1.0
    return jnp.where(u == 1.0, x,
                     jnp.where(um1 == -1.0, -1.0, um1 * x / jnp.log(u)))


def _log1mexp(x):
    mask = (-math.log(2.0)) < x
    return jnp.where(mask, jnp.log(-_expm1(x)), jnp.log1p(-jnp.exp(x)))


def _logaddexp_c(x1, x2):
    delta = jnp.where(x1 == x2, 0.0, x1 - x2)
    return jnp.maximum(x1, x2) + jax.nn.softplus(-jnp.abs(delta))


def _tc_dp_body(logits_t_ref, u_ref, d_ref, lp_ref, lq_ref):
    neg_inf = jnp.float32(-jnp.inf)

    lp = jnp.minimum(jax.nn.log_sigmoid(logits_t_ref[...]), -1e-07)
    lp_ref[...] = lp
    lq_ref[...] = _log1mexp(lp)

    rows = jax.lax.broadcasted_iota(jnp.int32, (_ROWS, _BSZ), 0)
    rows_valid = (rows >= 1) & (rows <= _K + 1)
    state0 = jnp.where(rows == 1, 0.0, neg_inf)

    def dp_step(t, state):
        lp_row = lp_ref[pl.ds(t, 1), :]
        lq_row = lq_ref[pl.ds(t, 1), :]
        s_lo = jnp.concatenate(
            [jnp.full((1, _BSZ), neg_inf, jnp.float32), state[:-1, :]],
            axis=0) + lp_row
        new = _logaddexp_c(s_lo, state + lq_row)
        p = s_lo - new
        u_row = u_ref[pl.ds(_N - 1 - t, 1), :]
        bit = (u_row < jnp.exp(p)).astype(jnp.int32)
        word = jnp.sum(jnp.where(rows_valid, bit << rows, 0), axis=0,
                       keepdims=True)
        d_ref[pl.ds(t, 1), :] = word
        return new

    jax.lax.fori_loop(0, _N, dp_step, state0)


def _tc_dp(logits_t, us):
    return pl.pallas_call(
        _tc_dp_body,
        out_shape=jax.ShapeDtypeStruct((_N, _BSZ), jnp.int32),
        in_specs=[
            pl.BlockSpec(memory_space=pltpu.VMEM),
            pl.BlockSpec(memory_space=pltpu.VMEM),
        ],
        out_specs=pl.BlockSpec(memory_space=pltpu.VMEM),
        scratch_shapes=[
            pltpu.VMEM((_N, _BSZ), jnp.float32),
            pltpu.VMEM((_N, _BSZ), jnp.float32),
        ],
    )(logits_t, us)


_SC_TILES = 32


def _sc_sampler_body(d_hbm, out_hbm, d_v, x_v):
    wid = lax.axis_index("s") * 2 + lax.axis_index("c")
    pltpu.sync_copy(d_hbm.at[:, wid], d_v)

    def step(t, j):
        r = _N - 1 - t
        w = d_v[r]
        bit = (w >> j) & 1
        x_v[r] = bit.astype(jnp.float32)
        return j - bit

    jax.lax.fori_loop(0, _N, step,
                      jnp.full((_SC_LANES,), _K + 1, jnp.int32))
    pltpu.sync_copy(x_v, out_hbm.at[:, wid])


def _sc_sampler(d):
    d3 = jnp.concatenate(
        [d, jnp.zeros((_N, (_SC_TILES - _SC_WORKERS) * _SC_LANES),
                      jnp.int32)], axis=1).reshape(_N, _SC_TILES, _SC_LANES)
    mesh = plsc.VectorSubcoreMesh(core_axis_name="c", subcore_axis_name="s",
                                  num_cores=2, num_subcores=16)
    run = pl.kernel(
        _sc_sampler_body,
        out_type=jax.ShapeDtypeStruct((_N, _SC_TILES, _SC_LANES),
                                      jnp.float32),
        mesh=mesh,
        scratch_types=[
            pltpu.VMEM((_N, _SC_LANES), jnp.int32),
            pltpu.VMEM((_N, _SC_LANES), jnp.float32),
        ],
        compiler_params=pltpu.CompilerParams(use_tc_tiling_on_sc=False),
    )
    out3 = run(d3)
    return out3[:, :_SC_WORKERS].reshape(_N, _BSZ)


def _uniforms():
    keys = jax.random.split(jax.random.key(42), _N)
    return jax.vmap(lambda k: jax.random.uniform(k, (_BSZ,)))(keys)


def kernel(logits):
    us = _uniforms()
    d = _tc_dp(logits.T, us)
    out_t = _sc_sampler(d)
    return out_t.T


# speedup vs baseline: 1.3856x; 1.3856x over previous
import functools
import math

import jax
import jax.numpy as jnp
from jax import lax
from jax.experimental import pallas as pl
from jax.experimental.pallas import tpu as pltpu
from jax.experimental.pallas import tpu_sc as plsc

_K = 8
_BSZ = 128
_N = 1000
_ROWS = 16
_SC_LANES = 16
_SC_WORKERS = _BSZ // _SC_LANES


def _expm1(x):
    u = jnp.exp(x)
    um1 = u - 1.0
    return jnp.where(u == 1.0, x,
                     jnp.where(um1 == -1.0, -1.0, um1 * x / jnp.log(u)))


def _log1mexp(x):
    mask = (-math.log(2.0)) < x
    return jnp.where(mask, jnp.log(-_expm1(x)), jnp.log1p(-jnp.exp(x)))


def _logaddexp_c(x1, x2):
    delta = jnp.where(x1 == x2, 0.0, x1 - x2)
    return jnp.maximum(x1, x2) + jax.nn.softplus(-jnp.abs(delta))


def _tc_dp_body(logits_t_ref, u_ref, d_ref, lp_ref, lq_ref):
    neg_inf = jnp.float32(-jnp.inf)

    lp = jnp.minimum(jax.nn.log_sigmoid(logits_t_ref[...]), -1e-07)
    lp_ref[...] = lp
    lq_ref[...] = _log1mexp(lp)

    rows = jax.lax.broadcasted_iota(jnp.int32, (_ROWS, _BSZ), 0)
    rows_valid = (rows >= 1) & (rows <= _K + 1)
    state0 = jnp.where(rows == 1, 0.0, neg_inf)

    def dp_step(t, state):
        lp_row = lp_ref[pl.ds(t, 1), :]
        lq_row = lq_ref[pl.ds(t, 1), :]
        s_lo = jnp.concatenate(
            [jnp.full((1, _BSZ), neg_inf, jnp.float32), state[:-1, :]],
            axis=0) + lp_row
        new = _logaddexp_c(s_lo, state + lq_row)
        p = s_lo - new
        u_row = u_ref[pl.ds(_N - 1 - t, 1), :]
        bit = (u_row < jnp.exp(p)).astype(jnp.int32)
        word = jnp.sum(jnp.where(rows_valid, bit << rows, 0), axis=0,
                       keepdims=True)
        d_ref[pl.ds(t, 1), :] = word
        return new

    jax.lax.fori_loop(0, _N, dp_step, state0)


def _tc_dp(logits_t, us):
    return pl.pallas_call(
        _tc_dp_body,
        out_shape=jax.ShapeDtypeStruct((_N, _BSZ), jnp.int32),
        in_specs=[
            pl.BlockSpec(memory_space=pltpu.VMEM),
            pl.BlockSpec(memory_space=pltpu.VMEM),
        ],
        out_specs=pl.BlockSpec(memory_space=pltpu.VMEM),
        scratch_shapes=[
            pltpu.VMEM((_N, _BSZ), jnp.float32),
            pltpu.VMEM((_N, _BSZ), jnp.float32),
        ],
    )(logits_t, us)


_SC_TILES = 32


def _sc_sampler_body(d_hbm, out_hbm, d_v, x_v):
    wid = lax.axis_index("s") * 2 + lax.axis_index("c")

    @pl.when(wid < _SC_WORKERS)
    def _():
        pltpu.sync_copy(d_hbm.at[:, wid], d_v)
        lanes = lax.iota(jnp.int32, _SC_LANES)

        def step(t, j):
            r = _N - 1 - t
            w = d_v[r]
            bit = (w >> j) & 1
            plsc.store_scatter(x_v, [lanes, jnp.full((_SC_LANES,), r,
                                                     jnp.int32)],
                               bit.astype(jnp.float32))
            return j - bit

        jax.lax.fori_loop(0, _N, step,
                          jnp.full((_SC_LANES,), _K + 1, jnp.int32))
        pltpu.sync_copy(
            x_v, out_hbm.at[pl.ds(wid * _SC_LANES, _SC_LANES), :])


def _sc_sampler(d):
    mesh = plsc.VectorSubcoreMesh(core_axis_name="c", subcore_axis_name="s",
                                  num_cores=2, num_subcores=16)
    run = pl.kernel(
        _sc_sampler_body,
        out_type=jax.ShapeDtypeStruct((_BSZ, _N), jnp.float32),
        mesh=mesh,
        scratch_types=[
            pltpu.VMEM((_N, _SC_LANES), jnp.int32),
            pltpu.VMEM((_SC_LANES, _N), jnp.float32),
        ],
        compiler_params=pltpu.CompilerParams(use_tc_tiling_on_sc=False,
                                             needs_layout_passes=False),
    )
    return run(d.reshape(_N, _SC_WORKERS, _SC_LANES))


def _uniforms():
    keys = jax.random.split(jax.random.key(42), _N)
    return jax.vmap(lambda k: jax.random.uniform(k, (_BSZ,)))(keys)


def kernel(logits):
    us = _uniforms()
    d = _tc_dp(logits.T, us)
    return _sc_sampler(d)


# speedup vs baseline: 1.5673x; 1.1312x over previous
import functools
import math

import jax
import jax.numpy as jnp
from jax import lax
from jax.experimental import pallas as pl
from jax.experimental.pallas import tpu as pltpu
from jax.experimental.pallas import tpu_sc as plsc

_K = 8
_BSZ = 128
_N = 1000
_ROWS = 16
_SC_LANES = 16
_SC_WORKERS = _BSZ // _SC_LANES


def _expm1(x):
    u = jnp.exp(x)
    um1 = u - 1.0
    return jnp.where(u == 1.0, x,
                     jnp.where(um1 == -1.0, -1.0, um1 * x / jnp.log(u)))


def _log1mexp(x):
    mask = (-math.log(2.0)) < x
    return jnp.where(mask, jnp.log(-_expm1(x)), jnp.log1p(-jnp.exp(x)))


def _logaddexp_c(x1, x2):
    delta = jnp.where(x1 == x2, 0.0, x1 - x2)
    return jnp.maximum(x1, x2) + jax.nn.softplus(-jnp.abs(delta))


def _tc_dp_body(logits_t_ref, u_ref, d_ref, lp_ref, lq_ref):
    neg_inf = jnp.float32(-jnp.inf)

    lp = jnp.minimum(jax.nn.log_sigmoid(logits_t_ref[...]), -1e-07)
    lp_ref[...] = lp
    lq_ref[...] = _log1mexp(lp)

    rows = jax.lax.broadcasted_iota(jnp.int32, (_ROWS, _BSZ), 0)
    rows_valid = (rows >= 1) & (rows <= _K + 1)
    state0 = jnp.where(rows == 1, 0.0, neg_inf)

    def dp_step(t, state):
        lp_row = lp_ref[pl.ds(t, 1), :]
        lq_row = lq_ref[pl.ds(t, 1), :]
        s_lo = jnp.concatenate(
            [jnp.full((1, _BSZ), neg_inf, jnp.float32), state[:-1, :]],
            axis=0) + lp_row
        new = _logaddexp_c(s_lo, state + lq_row)
        p = s_lo - new
        u_row = u_ref[pl.ds(_N - 1 - t, 1), :]
        bit = (u_row < jnp.exp(p)).astype(jnp.int32)
        word = jnp.sum(jnp.where(rows_valid, bit << rows, 0), axis=0,
                       keepdims=True)
        d_ref[pl.ds(t, 1), :] = word
        return new

    jax.lax.fori_loop(0, _N, dp_step, state0)


def _tc_dp(logits_t, us):
    return pl.pallas_call(
        _tc_dp_body,
        out_shape=jax.ShapeDtypeStruct((_N, _BSZ), jnp.int32),
        in_specs=[
            pl.BlockSpec(memory_space=pltpu.VMEM),
            pl.BlockSpec(memory_space=pltpu.VMEM),
        ],
        out_specs=pl.BlockSpec(memory_space=pltpu.VMEM),
        scratch_shapes=[
            pltpu.VMEM((_N, _BSZ), jnp.float32),
            pltpu.VMEM((_N, _BSZ), jnp.float32),
        ],
    )(logits_t, us)


_SC_TILES = 32


def _sc_sampler_body(d_hbm, out_hbm, d_v, x_v):
    wid = lax.axis_index("s") * 2 + lax.axis_index("c")

    @pl.when(wid < _SC_WORKERS)
    def _():
        pltpu.sync_copy(d_hbm.at[:, pl.ds(wid * _SC_LANES, _SC_LANES)], d_v)

        def step(t, j):
            r = _N - 1 - t
            w = d_v[r]
            bit = (w >> j) & 1
            x_v[r] = bit.astype(jnp.float32)
            return j - bit

        jax.lax.fori_loop(0, _N, step,
                          jnp.full((_SC_LANES,), _K + 1, jnp.int32))
        pltpu.sync_copy(
            x_v, out_hbm.at[:, pl.ds(wid * _SC_LANES, _SC_LANES)])


def _sc_sampler(d):
    mesh = plsc.VectorSubcoreMesh(core_axis_name="c", subcore_axis_name="s",
                                  num_cores=2, num_subcores=16)
    run = pl.kernel(
        _sc_sampler_body,
        out_type=jax.ShapeDtypeStruct((_N, _BSZ), jnp.float32),
        mesh=mesh,
        scratch_types=[
            pltpu.VMEM((_N, _SC_LANES), jnp.int32),
            pltpu.VMEM((_N, _SC_LANES), jnp.float32),
        ],
        compiler_params=pltpu.CompilerParams(use_tc_tiling_on_sc=False,
                                             needs_layout_passes=False),
    )
    return run(d)


def _uniforms():
    keys = jax.random.split(jax.random.key(42), _N)
    return jax.vmap(lambda k: jax.random.uniform(k, (_BSZ,)))(keys)


def kernel(logits):
    us = _uniforms()
    d = _tc_dp(logits.T, us)
    return _sc_sampler(d).T


# speedup vs baseline: 1.9009x; 1.2128x over previous
import functools
import math

import jax
import jax.numpy as jnp
from jax import lax
from jax.experimental import pallas as pl
from jax.experimental.pallas import tpu as pltpu
from jax.experimental.pallas import tpu_sc as plsc

_K = 8
_BSZ = 128
_N = 1000
_ROWS = 16
_SC_LANES = 16
_SC_WORKERS = _BSZ // _SC_LANES


def _expm1(x):
    u = jnp.exp(x)
    um1 = u - 1.0
    return jnp.where(u == 1.0, x,
                     jnp.where(um1 == -1.0, -1.0, um1 * x / jnp.log(u)))


def _log1mexp(x):
    mask = (-math.log(2.0)) < x
    return jnp.where(mask, jnp.log(-_expm1(x)), jnp.log1p(-jnp.exp(x)))


def _logaddexp_c(x1, x2):
    delta = jnp.where(x1 == x2, 0.0, x1 - x2)
    return jnp.maximum(x1, x2) + jax.nn.softplus(-jnp.abs(delta))


_CHUNK = 8


def _tc_dp_body(logits_t_ref, urev_ref, d_ref, lp_ref, lq_ref, a_ref):
    neg_inf = jnp.float32(-jnp.inf)

    lp = jnp.minimum(jax.nn.log_sigmoid(logits_t_ref[...]), -1e-07)
    lp_ref[...] = lp
    lq_ref[...] = _log1mexp(lp)

    rows = jax.lax.broadcasted_iota(jnp.int32, (_ROWS, _BSZ), 0)
    state0 = jnp.where(rows == 1, 0.0, neg_inf)
    a_ref[0] = state0

    def dp_step(t, carry):
        state, lp_row, lq_row = carry
        s_lo = jnp.concatenate(
            [jnp.full((1, _BSZ), neg_inf, jnp.float32), state[:-1, :]],
            axis=0) + lp_row
        new = _logaddexp_c(s_lo, state + lq_row)
        a_ref[pl.ds(t + 1, 1)] = new[None]
        tn = jnp.minimum(t + 1, _N - 1)
        return new, lp_ref[pl.ds(tn, 1), :], lq_ref[pl.ds(tn, 1), :]

    jax.lax.fori_loop(
        0, _N, dp_step,
        (state0, lp_ref[pl.ds(0, 1), :], lq_ref[pl.ds(0, 1), :]))

    rows3 = jax.lax.broadcasted_iota(jnp.int32, (_CHUNK, _ROWS, _BSZ), 1)
    rows3_valid = (rows3 >= 1) & (rows3 <= _K + 1)

    def d_chunk(c, _):
        t = c * _CHUNK
        a_lo = a_ref[pl.ds(t, _CHUNK)]
        a_hi = a_ref[pl.ds(t + 1, _CHUNK)]
        lp_c = lp_ref[pl.ds(t, _CHUNK), :][:, None, :]
        s_lo = jnp.concatenate(
            [jnp.full((_CHUNK, 1, _BSZ), neg_inf, jnp.float32),
             a_lo[:, :-1, :]], axis=1) + lp_c
        p = s_lo - a_hi
        u_c = urev_ref[pl.ds(t, _CHUNK), :][:, None, :]
        bit = (u_c < jnp.exp(p)).astype(jnp.int32)
        word = jnp.sum(jnp.where(rows3_valid, bit << rows3, 0), axis=1)
        d_ref[pl.ds(t, _CHUNK), :] = word
        return _

    jax.lax.fori_loop(0, _N // _CHUNK, d_chunk, 0)


def _tc_dp(logits_t, us_rev):
    return pl.pallas_call(
        _tc_dp_body,
        out_shape=jax.ShapeDtypeStruct((_N, _BSZ), jnp.int32),
        in_specs=[
            pl.BlockSpec(memory_space=pltpu.VMEM),
            pl.BlockSpec(memory_space=pltpu.VMEM),
        ],
        out_specs=pl.BlockSpec(memory_space=pltpu.VMEM),
        scratch_shapes=[
            pltpu.VMEM((_N, _BSZ), jnp.float32),
            pltpu.VMEM((_N, _BSZ), jnp.float32),
            pltpu.VMEM((_N + 1, _ROWS, _BSZ), jnp.float32),
        ],
    )(logits_t, us_rev)


_SC_TILES = 32


def _sc_sampler_body(d_hbm, out_hbm, d_v, x_v):
    wid = lax.axis_index("s") * 2 + lax.axis_index("c")

    @pl.when(wid < _SC_WORKERS)
    def _():
        pltpu.sync_copy(d_hbm.at[:, pl.ds(wid * _SC_LANES, _SC_LANES)], d_v)

        def step(t, j):
            r = _N - 1 - t
            w = d_v[r]
            bit = (w >> j) & 1
            x_v[r] = bit.astype(jnp.float32)
            return j - bit

        jax.lax.fori_loop(0, _N, step,
                          jnp.full((_SC_LANES,), _K + 1, jnp.int32))
        pltpu.sync_copy(
            x_v, out_hbm.at[:, pl.ds(wid * _SC_LANES, _SC_LANES)])


def _sc_sampler(d):
    mesh = plsc.VectorSubcoreMesh(core_axis_name="c", subcore_axis_name="s",
                                  num_cores=2, num_subcores=16)
    run = pl.kernel(
        _sc_sampler_body,
        out_type=jax.ShapeDtypeStruct((_N, _BSZ), jnp.float32),
        mesh=mesh,
        scratch_types=[
            pltpu.VMEM((_N, _SC_LANES), jnp.int32),
            pltpu.VMEM((_N, _SC_LANES), jnp.float32),
        ],
        compiler_params=pltpu.CompilerParams(use_tc_tiling_on_sc=False,
                                             needs_layout_passes=False),
    )
    return run(d)


def _uniforms():
    keys = jax.random.split(jax.random.key(42), _N)
    return jax.vmap(lambda k: jax.random.uniform(k, (_BSZ,)))(keys)


def kernel(logits):
    us = _uniforms()
    d = _tc_dp(logits.T, us[::-1])
    return _sc_sampler(d).T


# speedup vs baseline: 1.9244x; 1.0124x over previous
import functools
import math

import jax
import jax.numpy as jnp
from jax import lax
from jax.experimental import pallas as pl
from jax.experimental.pallas import tpu as pltpu
from jax.experimental.pallas import tpu_sc as plsc

_K = 8
_BSZ = 128
_N = 1000
_ROWS = 16
_SC_LANES = 16
_SC_WORKERS = _BSZ // _SC_LANES


def _expm1(x):
    u = jnp.exp(x)
    um1 = u - 1.0
    return jnp.where(u == 1.0, x,
                     jnp.where(um1 == -1.0, -1.0, um1 * x / jnp.log(u)))


def _log1mexp(x):
    mask = (-math.log(2.0)) < x
    return jnp.where(mask, jnp.log(-_expm1(x)), jnp.log1p(-jnp.exp(x)))


def _logaddexp_c(x1, x2):
    delta = jnp.where(x1 == x2, 0.0, x1 - x2)
    return jnp.maximum(x1, x2) + jax.nn.softplus(-jnp.abs(delta))


_CHUNK = 16


def _tc_dp_body(logits_t_ref, urev_ref, d_ref, lp_ref, lq_ref, a_ref):
    neg_inf = jnp.float32(-jnp.inf)

    lp = jnp.minimum(jax.nn.log_sigmoid(logits_t_ref[...]), -1e-07)
    lp_ref[...] = lp
    lq_ref[...] = _log1mexp(lp)

    rows = jax.lax.broadcasted_iota(jnp.int32, (_ROWS, _BSZ), 0)
    state0 = jnp.where(rows == 1, 0.0, neg_inf)
    a_ref[0] = state0

    def dp_step(t, carry):
        state, lp_row, lq_row = carry
        s_lo = jnp.concatenate(
            [jnp.full((1, _BSZ), neg_inf, jnp.float32), state[:-1, :]],
            axis=0) + lp_row
        new = _logaddexp_c(s_lo, state + lq_row)
        a_ref[pl.ds(t + 1, 1)] = new[None]
        tn = jnp.minimum(t + 1, _N - 1)
        return new, lp_ref[pl.ds(tn, 1), :], lq_ref[pl.ds(tn, 1), :]

    jax.lax.fori_loop(
        0, _N, dp_step,
        (state0, lp_ref[pl.ds(0, 1), :], lq_ref[pl.ds(0, 1), :]),
        unroll=2)

    rows3 = jax.lax.broadcasted_iota(jnp.int32, (_CHUNK, _ROWS, _BSZ), 1)
    rows3_valid = (rows3 >= 1) & (rows3 <= _K + 1)

    def d_chunk(c, _):
        t = c * _CHUNK
        a_lo = a_ref[pl.ds(t, _CHUNK)]
        a_hi = a_ref[pl.ds(t + 1, _CHUNK)]
        lp_c = lp_ref[pl.ds(t, _CHUNK), :][:, None, :]
        s_lo = jnp.concatenate(
            [jnp.full((_CHUNK, 1, _BSZ), neg_inf, jnp.float32),
             a_lo[:, :-1, :]], axis=1) + lp_c
        p = s_lo - a_hi
        u_c = urev_ref[pl.ds(t, _CHUNK), :][:, None, :]
        bit = (u_c < jnp.exp(p)).astype(jnp.int32)
        word = jnp.sum(jnp.where(rows3_valid, bit << rows3, 0), axis=1)
        d_ref[pl.ds(t, _CHUNK), :] = word
        return _

    jax.lax.fori_loop(0, _N // _CHUNK, d_chunk, 0)


def _tc_dp(logits_t, us_rev):
    return pl.pallas_call(
        _tc_dp_body,
        out_shape=jax.ShapeDtypeStruct((_N, _BSZ), jnp.int32),
        in_specs=[
            pl.BlockSpec(memory_space=pltpu.VMEM),
            pl.BlockSpec(memory_space=pltpu.VMEM),
        ],
        out_specs=pl.BlockSpec(memory_space=pltpu.VMEM),
        scratch_shapes=[
            pltpu.VMEM((_N, _BSZ), jnp.float32),
            pltpu.VMEM((_N, _BSZ), jnp.float32),
            pltpu.VMEM((_N + 1, _ROWS, _BSZ), jnp.float32),
        ],
    )(logits_t, us_rev)


_SC_TILES = 32


def _sc_sampler_body(d_hbm, out_hbm, d_v, x_v):
    wid = lax.axis_index("s") * 2 + lax.axis_index("c")

    @pl.when(wid < _SC_WORKERS)
    def _():
        pltpu.sync_copy(d_hbm.at[:, pl.ds(wid * _SC_LANES, _SC_LANES)], d_v)

        def step(t, j):
            r = _N - 1 - t
            w = d_v[r]
            bit = (w >> j) & 1
            x_v[r] = bit.astype(jnp.float32)
            return j - bit

        jax.lax.fori_loop(0, _N, step,
                          jnp.full((_SC_LANES,), _K + 1, jnp.int32))
        pltpu.sync_copy(
            x_v, out_hbm.at[:, pl.ds(wid * _SC_LANES, _SC_LANES)])


def _sc_sampler(d):
    mesh = plsc.VectorSubcoreMesh(core_axis_name="c", subcore_axis_name="s",
                                  num_cores=2, num_subcores=16)
    run = pl.kernel(
        _sc_sampler_body,
        out_type=jax.ShapeDtypeStruct((_N, _BSZ), jnp.float32),
        mesh=mesh,
        scratch_types=[
            pltpu.VMEM((_N, _SC_LANES), jnp.int32),
            pltpu.VMEM((_N, _SC_LANES), jnp.float32),
        ],
        compiler_params=pltpu.CompilerParams(use_tc_tiling_on_sc=False,
                                             needs_layout_passes=False),
    )
    return run(d)


def _uniforms():
    keys = jax.random.split(jax.random.key(42), _N)
    return jax.vmap(lambda k: jax.random.uniform(k, (_BSZ,)))(keys)


def kernel(logits):
    us = _uniforms()
    d = _tc_dp(logits.T, us[::-1])
    return _sc_sampler(d).T


# speedup vs baseline: 1.9259x; 1.0008x over previous
import functools
import math

import jax
import jax.numpy as jnp
from jax import lax
from jax.experimental import pallas as pl
from jax.experimental.pallas import tpu as pltpu
from jax.experimental.pallas import tpu_sc as plsc

_K = 8
_BSZ = 128
_N = 1000
_ROWS = 16
_SC_LANES = 16
_SC_WORKERS = _BSZ // _SC_LANES


def _expm1(x):
    u = jnp.exp(x)
    um1 = u - 1.0
    return jnp.where(u == 1.0, x,
                     jnp.where(um1 == -1.0, -1.0, um1 * x / jnp.log(u)))


def _log1mexp(x):
    mask = (-math.log(2.0)) < x
    return jnp.where(mask, jnp.log(-_expm1(x)), jnp.log1p(-jnp.exp(x)))


def _logaddexp_c(x1, x2):
    delta = jnp.where(x1 == x2, 0.0, x1 - x2)
    return jnp.maximum(x1, x2) + jax.nn.softplus(-jnp.abs(delta))


_CHUNK = 8


def _tc_dp_body(logits_t_ref, urev_ref, d_ref, lp_ref, lq_ref, a_ref):
    neg_inf = jnp.float32(-jnp.inf)

    lp = jnp.minimum(jax.nn.log_sigmoid(logits_t_ref[...]), -1e-07)
    lp_ref[...] = lp
    lq_ref[...] = _log1mexp(lp)

    rows = jax.lax.broadcasted_iota(jnp.int32, (_ROWS, _BSZ), 0)
    state0 = jnp.where(rows == 1, 0.0, neg_inf)
    a_ref[0] = state0

    def dp_step(t, carry):
        state, lp_row, lq_row = carry
        s_lo = jnp.concatenate(
            [jnp.full((1, _BSZ), neg_inf, jnp.float32), state[:-1, :]],
            axis=0) + lp_row
        new = _logaddexp_c(s_lo, state + lq_row)
        a_ref[pl.ds(t + 1, 1)] = new[None]
        tn = jnp.minimum(t + 1, _N - 1)
        return new, lp_ref[pl.ds(tn, 1), :], lq_ref[pl.ds(tn, 1), :]

    jax.lax.fori_loop(
        0, _N, dp_step,
        (state0, lp_ref[pl.ds(0, 1), :], lq_ref[pl.ds(0, 1), :]),
        unroll=2)

    rows3 = jax.lax.broadcasted_iota(jnp.int32, (_CHUNK, _ROWS, _BSZ), 1)
    rows3_valid = (rows3 >= 1) & (rows3 <= _K + 1)

    def d_chunk(c, _):
        t = c * _CHUNK
        a_lo = a_ref[pl.ds(t, _CHUNK)]
        a_hi = a_ref[pl.ds(t + 1, _CHUNK)]
        lp_c = lp_ref[pl.ds(t, _CHUNK), :][:, None, :]
        s_lo = jnp.concatenate(
            [jnp.full((_CHUNK, 1, _BSZ), neg_inf, jnp.float32),
             a_lo[:, :-1, :]], axis=1) + lp_c
        p = s_lo - a_hi
        u_c = urev_ref[pl.ds(t, _CHUNK), :][:, None, :]
        bit = (u_c < jnp.exp(p)).astype(jnp.int32)
        word = jnp.sum(jnp.where(rows3_valid, bit << rows3, 0), axis=1)
        d_ref[pl.ds(t, _CHUNK), :] = word
        return _

    jax.lax.fori_loop(0, _N // _CHUNK, d_chunk, 0)


def _tc_dp(logits_t, us_rev):
    return pl.pallas_call(
        _tc_dp_body,
        out_shape=jax.ShapeDtypeStruct((_N, _BSZ), jnp.int32),
        in_specs=[
            pl.BlockSpec(memory_space=pltpu.VMEM),
            pl.BlockSpec(memory_space=pltpu.VMEM),
        ],
        out_specs=pl.BlockSpec(memory_space=pltpu.VMEM),
        scratch_shapes=[
            pltpu.VMEM((_N, _BSZ), jnp.float32),
            pltpu.VMEM((_N, _BSZ), jnp.float32),
            pltpu.VMEM((_N + 1, _ROWS, _BSZ), jnp.float32),
        ],
    )(logits_t, us_rev)


_SC_TILES = 32


def _sc_sampler_body(d_hbm, out_hbm, d_v, x_v):
    wid = lax.axis_index("s") * 2 + lax.axis_index("c")

    @pl.when(wid < _SC_WORKERS)
    def _():
        pltpu.sync_copy(d_hbm.at[:, pl.ds(wid * _SC_LANES, _SC_LANES)], d_v)

        def step(t, j):
            r = _N - 1 - t
            w = d_v[r]
            bit = (w >> j) & 1
            x_v[r] = bit.astype(jnp.float32)
            return j - bit

        jax.lax.fori_loop(0, _N, step,
                          jnp.full((_SC_LANES,), _K + 1, jnp.int32))
        pltpu.sync_copy(
            x_v, out_hbm.at[:, pl.ds(wid * _SC_LANES, _SC_LANES)])


def _sc_sampler(d):
    mesh = plsc.VectorSubcoreMesh(core_axis_name="c", subcore_axis_name="s",
                                  num_cores=2, num_subcores=16)
    run = pl.kernel(
        _sc_sampler_body,
        out_type=jax.ShapeDtypeStruct((_N, _BSZ), jnp.float32),
        mesh=mesh,
        scratch_types=[
            pltpu.VMEM((_N, _SC_LANES), jnp.int32),
            pltpu.VMEM((_N, _SC_LANES), jnp.float32),
        ],
        compiler_params=pltpu.CompilerParams(use_tc_tiling_on_sc=False,
                                             needs_layout_passes=False),
    )
    return run(d)


def _uniforms():
    keys = jax.random.split(jax.random.key(42), _N)
    return jax.vmap(lambda k: jax.random.uniform(k, (_BSZ,)))(keys)


def kernel(logits):
    us = _uniforms()
    d = _tc_dp(logits.T, us[::-1])
    return _sc_sampler(d).T


# speedup vs baseline: 1.9373x; 1.0059x over previous
import functools
import math

import jax
import jax.numpy as jnp
from jax import lax
from jax.experimental import pallas as pl
from jax.experimental.pallas import tpu as pltpu
from jax.experimental.pallas import tpu_sc as plsc

_K = 8
_BSZ = 128
_N = 1000
_ROWS = 16
_SC_LANES = 16
_SC_WORKERS = _BSZ // _SC_LANES


def _expm1(x):
    u = jnp.exp(x)
    um1 = u - 1.0
    return jnp.where(u == 1.0, x,
                     jnp.where(um1 == -1.0, -1.0, um1 * x / jnp.log(u)))


def _log1mexp(x):
    mask = (-math.log(2.0)) < x
    return jnp.where(mask, jnp.log(-_expm1(x)), jnp.log1p(-jnp.exp(x)))


def _logaddexp_c(x1, x2):
    delta = jnp.where(x1 == x2, 0.0, x1 - x2)
    return jnp.maximum(x1, x2) + jax.nn.softplus(-jnp.abs(delta))


def _logaddexp_dp(x1, x2):
    c = jnp.float32(-3e38)
    m1 = jnp.maximum(x1, c)
    m2 = jnp.maximum(x2, c)
    negabs = jnp.minimum(m1, m2) - jnp.maximum(m1, m2)
    return jnp.maximum(x1, x2) + jax.nn.softplus(negabs)


_CHUNK = 8


def _tc_dp_body(logits_t_ref, urev_ref, d_ref, lp_ref, lq_ref, a_ref):
    neg_inf = jnp.float32(-jnp.inf)

    lp = jnp.minimum(jax.nn.log_sigmoid(logits_t_ref[...]), -1e-07)
    lp_ref[...] = lp
    lq_ref[...] = _log1mexp(lp)

    rows = jax.lax.broadcasted_iota(jnp.int32, (_ROWS, _BSZ), 0)
    state0 = jnp.where(rows == 1, 0.0, neg_inf)
    a_ref[0] = state0

    def dp_step(t, carry):
        state, lp_row, lq_row = carry
        s_lo = jnp.concatenate(
            [jnp.full((1, _BSZ), neg_inf, jnp.float32), state[:-1, :]],
            axis=0) + lp_row
        new = _logaddexp_dp(s_lo, state + lq_row)
        a_ref[pl.ds(t + 1, 1)] = new[None]
        tn = jnp.minimum(t + 1, _N - 1)
        return new, lp_ref[pl.ds(tn, 1), :], lq_ref[pl.ds(tn, 1), :]

    jax.lax.fori_loop(
        0, _N, dp_step,
        (state0, lp_ref[pl.ds(0, 1), :], lq_ref[pl.ds(0, 1), :]),
        unroll=2)

    rows3 = jax.lax.broadcasted_iota(jnp.int32, (_CHUNK, _ROWS, _BSZ), 1)
    rows3_valid = (rows3 >= 1) & (rows3 <= _K + 1)

    def d_chunk(c, _):
        t = c * _CHUNK
        a_lo = a_ref[pl.ds(t, _CHUNK)]
        a_hi = a_ref[pl.ds(t + 1, _CHUNK)]
        lp_c = lp_ref[pl.ds(t, _CHUNK), :][:, None, :]
        s_lo = jnp.concatenate(
            [jnp.full((_CHUNK, 1, _BSZ), neg_inf, jnp.float32),
             a_lo[:, :-1, :]], axis=1) + lp_c
        p = s_lo - a_hi
        u_c = urev_ref[pl.ds(t, _CHUNK), :][:, None, :]
        bit = (u_c < jnp.exp(p)).astype(jnp.int32)
        word = jnp.sum(jnp.where(rows3_valid, bit << rows3, 0), axis=1)
        d_ref[pl.ds(t, _CHUNK), :] = word
        return _

    jax.lax.fori_loop(0, _N // _CHUNK, d_chunk, 0)


def _tc_dp(logits_t, us_rev):
    return pl.pallas_call(
        _tc_dp_body,
        out_shape=jax.ShapeDtypeStruct((_N, _BSZ), jnp.int32),
        in_specs=[
            pl.BlockSpec(memory_space=pltpu.VMEM),
            pl.BlockSpec(memory_space=pltpu.VMEM),
        ],
        out_specs=pl.BlockSpec(memory_space=pltpu.VMEM),
        scratch_shapes=[
            pltpu.VMEM((_N, _BSZ), jnp.float32),
            pltpu.VMEM((_N, _BSZ), jnp.float32),
            pltpu.VMEM((_N + 1, _ROWS, _BSZ), jnp.float32),
        ],
    )(logits_t, us_rev)


_SC_TILES = 32


def _sc_sampler_body(d_hbm, out_hbm, d_v, x_v):
    wid = lax.axis_index("s") * 2 + lax.axis_index("c")

    @pl.when(wid < _SC_WORKERS)
    def _():
        pltpu.sync_copy(d_hbm.at[:, pl.ds(wid * _SC_LANES, _SC_LANES)], d_v)

        def step(t, j):
            r = _N - 1 - t
            w = d_v[r]
            bit = (w >> j) & 1
            x_v[r] = bit.astype(jnp.float32)
            return j - bit

        jax.lax.fori_loop(0, _N, step,
                          jnp.full((_SC_LANES,), _K + 1, jnp.int32))
        pltpu.sync_copy(
            x_v, out_hbm.at[:, pl.ds(wid * _SC_LANES, _SC_LANES)])


def _sc_sampler(d):
    mesh = plsc.VectorSubcoreMesh(core_axis_name="c", subcore_axis_name="s",
                                  num_cores=2, num_subcores=16)
    run = pl.kernel(
        _sc_sampler_body,
        out_type=jax.ShapeDtypeStruct((_N, _BSZ), jnp.float32),
        mesh=mesh,
        scratch_types=[
            pltpu.VMEM((_N, _SC_LANES), jnp.int32),
            pltpu.VMEM((_N, _SC_LANES), jnp.float32),
        ],
        compiler_params=pltpu.CompilerParams(use_tc_tiling_on_sc=False,
                                             needs_layout_passes=False),
    )
    return run(d)


def _uniforms():
    keys = jax.random.split(jax.random.key(42), _N)
    return jax.vmap(lambda k: jax.random.uniform(k, (_BSZ,)))(keys)


def kernel(logits):
    us = _uniforms()
    d = _tc_dp(logits.T, us[::-1])
    return _sc_sampler(d).T


# speedup vs baseline: 2.0608x; 1.0637x over previous
import functools
import math

import jax
import jax.numpy as jnp
from jax import lax
from jax.experimental import pallas as pl
from jax.experimental.pallas import tpu as pltpu
from jax.experimental.pallas import tpu_sc as plsc

_K = 8
_BSZ = 128
_N = 1000
_ROWS = 16
_SC_LANES = 16
_SC_WORKERS = _BSZ // _SC_LANES


def _expm1(x):
    u = jnp.exp(x)
    um1 = u - 1.0
    return jnp.where(u == 1.0, x,
                     jnp.where(um1 == -1.0, -1.0, um1 * x / jnp.log(u)))


def _log1mexp(x):
    mask = (-math.log(2.0)) < x
    return jnp.where(mask, jnp.log(-_expm1(x)), jnp.log1p(-jnp.exp(x)))


def _logaddexp_c(x1, x2):
    delta = jnp.where(x1 == x2, 0.0, x1 - x2)
    return jnp.maximum(x1, x2) + jax.nn.softplus(-jnp.abs(delta))


def _logaddexp_dp(x1, x2):
    c = jnp.float32(-3e38)
    m1 = jnp.maximum(x1, c)
    m2 = jnp.maximum(x2, c)
    negabs = jnp.minimum(m1, m2) - jnp.maximum(m1, m2)
    return jnp.maximum(x1, x2) + jnp.log1p(jnp.exp(negabs))


_CHUNK = 8


def _tc_dp_body(logits_t_ref, urev_ref, d_ref, lp_ref, lq_ref, a_ref):
    neg_inf = jnp.float32(-jnp.inf)

    lp = jnp.minimum(jax.nn.log_sigmoid(logits_t_ref[...]), -1e-07)
    lp_ref[...] = lp
    lq_ref[...] = _log1mexp(lp)

    rows = jax.lax.broadcasted_iota(jnp.int32, (_ROWS, _BSZ), 0)
    state0 = jnp.where(rows == 1, 0.0, neg_inf)
    a_ref[0] = state0

    def dp_step(t, carry):
        state, lp_row, lq_row = carry
        s_lo = jnp.concatenate(
            [jnp.full((1, _BSZ), neg_inf, jnp.float32), state[:-1, :]],
            axis=0) + lp_row
        new = _logaddexp_dp(s_lo, state + lq_row)
        a_ref[pl.ds(t + 1, 1)] = new[None]
        tn = jnp.minimum(t + 1, _N - 1)
        return new, lp_ref[pl.ds(tn, 1), :], lq_ref[pl.ds(tn, 1), :]

    jax.lax.fori_loop(
        0, _N, dp_step,
        (state0, lp_ref[pl.ds(0, 1), :], lq_ref[pl.ds(0, 1), :]),
        unroll=2)

    rows3 = jax.lax.broadcasted_iota(jnp.int32, (_CHUNK, _ROWS, _BSZ), 1)
    rows3_valid = (rows3 >= 1) & (rows3 <= _K + 1)

    def d_chunk(c, _):
        t = c * _CHUNK
        a_lo = a_ref[pl.ds(t, _CHUNK)]
        a_hi = a_ref[pl.ds(t + 1, _CHUNK)]
        lp_c = lp_ref[pl.ds(t, _CHUNK), :][:, None, :]
        s_lo = jnp.concatenate(
            [jnp.full((_CHUNK, 1, _BSZ), neg_inf, jnp.float32),
             a_lo[:, :-1, :]], axis=1) + lp_c
        p = s_lo - a_hi
        u_c = urev_ref[pl.ds(t, _CHUNK), :][:, None, :]
        bit = (u_c < jnp.exp(p)).astype(jnp.int32)
        word = jnp.sum(jnp.where(rows3_valid, bit << rows3, 0), axis=1)
        d_ref[pl.ds(t, _CHUNK), :] = word
        return _

    jax.lax.fori_loop(0, _N // _CHUNK, d_chunk, 0)


def _tc_dp(logits_t, us_rev):
    return pl.pallas_call(
        _tc_dp_body,
        out_shape=jax.ShapeDtypeStruct((_N, _BSZ), jnp.int32),
        in_specs=[
            pl.BlockSpec(memory_space=pltpu.VMEM),
            pl.BlockSpec(memory_space=pltpu.VMEM),
        ],
        out_specs=pl.BlockSpec(memory_space=pltpu.VMEM),
        scratch_shapes=[
            pltpu.VMEM((_N, _BSZ), jnp.float32),
            pltpu.VMEM((_N, _BSZ), jnp.float32),
            pltpu.VMEM((_N + 1, _ROWS, _BSZ), jnp.float32),
        ],
    )(logits_t, us_rev)


_SC_TILES = 32


def _sc_sampler_body(d_hbm, out_hbm, d_v, x_v):
    wid = lax.axis_index("s") * 2 + lax.axis_index("c")

    @pl.when(wid < _SC_WORKERS)
    def _():
        pltpu.sync_copy(d_hbm.at[:, pl.ds(wid * _SC_LANES, _SC_LANES)], d_v)

        def step(t, j):
            r = _N - 1 - t
            w = d_v[r]
            bit = (w >> j) & 1
            x_v[r] = bit.astype(jnp.float32)
            return j - bit

        jax.lax.fori_loop(0, _N, step,
                          jnp.full((_SC_LANES,), _K + 1, jnp.int32))
        pltpu.sync_copy(
            x_v, out_hbm.at[:, pl.ds(wid * _SC_LANES, _SC_LANES)])


def _sc_sampler(d):
    mesh = plsc.VectorSubcoreMesh(core_axis_name="c", subcore_axis_name="s",
                                  num_cores=2, num_subcores=16)
    run = pl.kernel(
        _sc_sampler_body,
        out_type=jax.ShapeDtypeStruct((_N, _BSZ), jnp.float32),
        mesh=mesh,
        scratch_types=[
            pltpu.VMEM((_N, _SC_LANES), jnp.int32),
            pltpu.VMEM((_N, _SC_LANES), jnp.float32),
        ],
        compiler_params=pltpu.CompilerParams(use_tc_tiling_on_sc=False,
                                             needs_layout_passes=False),
    )
    return run(d)


def _uniforms():
    keys = jax.random.split(jax.random.key(42), _N)
    return jax.vmap(lambda k: jax.random.uniform(k, (_BSZ,)))(keys)


def kernel(logits):
    us = _uniforms()
    d = _tc_dp(logits.T, us[::-1])
    return _sc_sampler(d).T


# speedup vs baseline: 2.1617x; 1.0490x over previous
import functools
import math

import jax
import jax.numpy as jnp
from jax import lax
from jax.experimental import pallas as pl
from jax.experimental.pallas import tpu as pltpu
from jax.experimental.pallas import tpu_sc as plsc

_K = 8
_BSZ = 128
_N = 1000
_ROWS = 16
_SC_LANES = 16
_SC_WORKERS = _BSZ // _SC_LANES


def _expm1(x):
    u = jnp.exp(x)
    um1 = u - 1.0
    return jnp.where(u == 1.0, x,
                     jnp.where(um1 == -1.0, -1.0, um1 * x / jnp.log(u)))


def _log1mexp(x):
    mask = (-math.log(2.0)) < x
    return jnp.where(mask, jnp.log(-_expm1(x)), jnp.log1p(-jnp.exp(x)))


def _logaddexp_c(x1, x2):
    delta = jnp.where(x1 == x2, 0.0, x1 - x2)
    return jnp.maximum(x1, x2) + jax.nn.softplus(-jnp.abs(delta))


def _logaddexp_dp(x1, x2):
    c = jnp.float32(-3e38)
    m1 = jnp.maximum(x1, c)
    m2 = jnp.maximum(x2, c)
    negabs = jnp.minimum(m1, m2) - jnp.maximum(m1, m2)
    return jnp.maximum(x1, x2) + jnp.log1p(jnp.exp(negabs))


_CHUNK = 8


def _tc_dp_body(logits_t_ref, urev_ref, d_ref, lp_ref, lq_ref, a_ref):
    neg_inf = jnp.float32(-jnp.inf)

    lp = jnp.minimum(jax.nn.log_sigmoid(logits_t_ref[...]), -1e-07)
    lp_ref[...] = lp
    lq_ref[...] = _log1mexp(lp)

    rows = jax.lax.broadcasted_iota(jnp.int32, (_ROWS, _BSZ), 0)
    state0 = jnp.where(rows == 1, 0.0, neg_inf)
    a_ref[0] = state0

    def dp_step(t, carry):
        state, lp_row, lq_row = carry
        s_lo = jnp.concatenate(
            [jnp.full((1, _BSZ), neg_inf, jnp.float32), state[:-1, :]],
            axis=0) + lp_row
        new = _logaddexp_dp(s_lo, state + lq_row)
        a_ref[pl.ds(t + 1, 1)] = new[None]
        tn = jnp.minimum(t + 1, _N - 1)
        return new, lp_ref[pl.ds(tn, 1), :], lq_ref[pl.ds(tn, 1), :]

    jax.lax.fori_loop(
        0, _N, dp_step,
        (state0, lp_ref[pl.ds(0, 1), :], lq_ref[pl.ds(0, 1), :]),
        unroll=2)

    rows3 = jax.lax.broadcasted_iota(jnp.int32, (_CHUNK, _ROWS, _BSZ), 1)
    rows3_valid = (rows3 >= 1) & (rows3 <= _K + 1)

    def d_chunk(c, _):
        t = c * _CHUNK
        a_lo = a_ref[pl.ds(t, _CHUNK)]
        a_hi = a_ref[pl.ds(t + 1, _CHUNK)]
        lp_c = lp_ref[pl.ds(t, _CHUNK), :][:, None, :]
        s_lo = jnp.concatenate(
            [jnp.full((_CHUNK, 1, _BSZ), neg_inf, jnp.float32),
             a_lo[:, :-1, :]], axis=1) + lp_c
        p = s_lo - a_hi
        u_c = urev_ref[pl.ds(t, _CHUNK), :][:, None, :]
        bit = (u_c < jnp.exp(p)).astype(jnp.int32)
        word = jnp.sum(jnp.where(rows3_valid, bit << rows3, 0), axis=1)
        d_ref[pl.ds(t, _CHUNK), :] = word
        return _

    jax.lax.fori_loop(0, _N // _CHUNK, d_chunk, 0)


def _tc_dp(logits_t, us_rev):
    return pl.pallas_call(
        _tc_dp_body,
        out_shape=jax.ShapeDtypeStruct((_N, _BSZ), jnp.int32),
        in_specs=[
            pl.BlockSpec(memory_space=pltpu.VMEM),
            pl.BlockSpec(memory_space=pltpu.VMEM),
        ],
        out_specs=pl.BlockSpec(memory_space=pltpu.VMEM),
        scratch_shapes=[
            pltpu.VMEM((_N, _BSZ), jnp.float32),
            pltpu.VMEM((_N, _BSZ), jnp.float32),
            pltpu.VMEM((_N + 1, _ROWS, _BSZ), jnp.float32),
        ],
    )(logits_t, us_rev)


_SC_TILES = 32


_NBLK = 4
_BLK = _N // _NBLK
_NHYP = _K + 1


def _sc_sampler_body(d_hbm, out_hbm, d_v, x_v, m_stage, m_all, maps_sh):
    c = lax.axis_index("c")
    s = lax.axis_index("s")
    g = c * 4 + s // 4
    b = s % 4
    lane0 = g * _SC_LANES
    row0 = b * _BLK
    pltpu.sync_copy(d_hbm.at[pl.ds(row0, _BLK), pl.ds(lane0, _SC_LANES)],
                    d_v)

    def hyp_step(t, js):
        w = d_v[_BLK - 1 - t]
        return tuple(j - ((w >> j) & 1) for j in js)

    js0 = tuple(jnp.full((_SC_LANES,), h, jnp.int32)
                for h in range(1, _NHYP + 1))
    js = jax.lax.fori_loop(0, _BLK, hyp_step, js0)
    for h in range(_NHYP):
        m_stage[h] = js[h]

    pltpu.sync_copy(m_stage, maps_sh.at[s])
    plsc.subcore_barrier()
    pltpu.sync_copy(maps_sh.at[pl.ds((s // 4) * 4, _NBLK)], m_all)
    lanes = lax.iota(jnp.int32, _SC_LANES)
    j = jnp.full((_SC_LANES,), _NHYP, jnp.int32)
    for bp in (3, 2, 1):
        mapped = plsc.load_gather(
            m_all, [jnp.full((_SC_LANES,), bp, jnp.int32), j - 1, lanes])
        j = jnp.where(bp > b, mapped, j)

    def step(t, j):
        r = _BLK - 1 - t
        w = d_v[r]
        bit = (w >> j) & 1
        x_v[r] = bit.astype(jnp.float32)
        return j - bit

    jax.lax.fori_loop(0, _BLK, step, j)
    pltpu.sync_copy(
        x_v, out_hbm.at[pl.ds(row0, _BLK), pl.ds(lane0, _SC_LANES)])


def _sc_sampler(d):
    mesh = plsc.VectorSubcoreMesh(core_axis_name="c", subcore_axis_name="s",
                                  num_cores=2, num_subcores=16)
    run = pl.kernel(
        _sc_sampler_body,
        out_type=jax.ShapeDtypeStruct((_N, _BSZ), jnp.float32),
        mesh=mesh,
        scratch_types=[
            pltpu.VMEM((_BLK, _SC_LANES), jnp.int32),
            pltpu.VMEM((_BLK, _SC_LANES), jnp.float32),
            pltpu.VMEM((_NHYP, _SC_LANES), jnp.int32),
            pltpu.VMEM((_NBLK, _NHYP, _SC_LANES), jnp.int32),
            pltpu.VMEM_SHARED((16, _NHYP, _SC_LANES), jnp.int32),
        ],
        compiler_params=pltpu.CompilerParams(use_tc_tiling_on_sc=False,
                                             needs_layout_passes=False),
    )
    return run(d)


def _uniforms():
    keys = jax.random.split(jax.random.key(42), _N)
    return jax.vmap(lambda k: jax.random.uniform(k, (_BSZ,)))(keys)


def kernel(logits):
    us = _uniforms()
    d = _tc_dp(logits.T, us[::-1])
    return _sc_sampler(d).T
